# baseline (speedup 1.0000x reference)
import jax
import jax.numpy as jnp
from jax import lax
from jax.experimental import pallas as pl
from jax.experimental.pallas import tpu as pltpu
from jax.experimental.pallas import tpu_sc as plsc

N_NODE = 50000
E = 800000
D_IN = 128
H = 64
HALF = H // 2
NC = 2
NS = 16
NW = NC * NS
IB = 100
KB = 2
EROWS = E // IB
RPS = EROWS // NS
NPAIR = RPS // (2 * KB)
ROWS_PT = N_NODE // NS
FCH = 100

CIB = 128
CROWS_T = E // CIB
CR = CROWS_T // NW
CTAIL = CROWS_T - NW * CR
CKB = 5

_sc_params = pltpu.CompilerParams(use_tc_tiling_on_sc=False,
                                  needs_layout_passes=False)
_sc_mesh = dict(core_axis_name="c", subcore_axis_name="s")


def _seg_round(h_lo, h_hi, src2d, dst2d, zrows, out_hbm, acc,
               srcA, dstA, srcB, dstB, rowsA, rowsB, gsA, gsB, c, s):
    pltpu.sync_copy(zrows, rowsA.at[0])
    for j in range(ROWS_PT // FCH):
        pltpu.sync_copy(rowsA.at[0],
                        acc.at[pl.ds(s * ROWS_PT + j * FCH, FCH)])
    pltpu.sync_copy(rowsA.at[0, pl.ds(0, 25)],
                    acc.at[pl.ds(s * ROWS_PT + 3100, 25)])
    plsc.subcore_barrier()

    def _accum(h_half):
        base = s * RPS

        def load_fire(row0, src_v, dst_v, rows_v, sem):
            pltpu.sync_copy(src2d.at[pl.ds(row0, KB)], src_v)
            pltpu.sync_copy(dst2d.at[pl.ds(row0, KB)], dst_v)
            for k in range(KB):
                pltpu.async_copy(h_half.at[src_v.at[k]], rows_v.at[k], sem)

        def wait_scatter(src_v, dst_v, rows_v, sem):
            for k in range(KB):
                pltpu.make_async_copy(h_half.at[src_v.at[k]],
                                      rows_v.at[k], sem).wait()
            for k in range(KB):
                pltpu.sync_copy(rows_v.at[k], acc.at[dst_v.at[k]], add=True)

        load_fire(base, srcA, dstA, rowsA, gsA)

        def step(j, carry):
            row0 = base + 2 * KB * j
            load_fire(row0 + KB, srcB, dstB, rowsB, gsB)
            wait_scatter(srcA, dstA, rowsA, gsA)

            @pl.when(j < NPAIR - 1)
            def _():
                load_fire(row0 + 2 * KB, srcA, dstA, rowsA, gsA)

            wait_scatter(srcB, dstB, rowsB, gsB)
            return carry
        lax.fori_loop(0, NPAIR, step, 0)

    @pl.when(c == 0)
    def _lo():
        _accum(h_lo)

    @pl.when(c == 1)
    def _hi():
        _accum(h_hi)

    plsc.subcore_barrier()

    for j in range(ROWS_PT // FCH):
        r0 = s * ROWS_PT + j * FCH
        pltpu.sync_copy(acc.at[pl.ds(r0, FCH)], rowsA.at[0])
        pltpu.sync_copy(rowsA.at[0], out_hbm.at[pl.ds(c * N_NODE + r0, FCH)])
    r0 = s * ROWS_PT + 3100
    pltpu.sync_copy(acc.at[pl.ds(r0, 25)], rowsA.at[0, pl.ds(0, 25)])
    pltpu.sync_copy(rowsA.at[0, pl.ds(0, 25)],
                    out_hbm.at[pl.ds(c * N_NODE + r0, 25)])


def _seg_scratch():
    return [
        pltpu.VMEM_SHARED((N_NODE, HALF), jnp.float32),
        pltpu.VMEM((KB, IB), jnp.int32),
        pltpu.VMEM((KB, IB), jnp.int32),
        pltpu.VMEM((KB, IB), jnp.int32),
        pltpu.VMEM((KB, IB), jnp.int32),
        pltpu.VMEM((KB, IB, HALF), jnp.float32),
        pltpu.VMEM((KB, IB, HALF), jnp.float32),
        pltpu.SemaphoreType.DMA,
        pltpu.SemaphoreType.DMA,
    ]


def _segsum2_body(hu_lo, hu_hi, ha_lo, ha_hi, srcr, dstr, srcv, dstv, zrows,
                  sum_r, sum_v, *scr):
    c = lax.axis_index("c")
    s = lax.axis_index("s")
    _seg_round(hu_lo, hu_hi, srcr, dstr, zrows, sum_r, *scr, c, s)
    _seg_round(ha_lo, ha_hi, srcv, dstv, zrows, sum_v, *scr, c, s)


def _segsum1_body(h_lo, h_hi, src2d, dst2d, zrows, sum_out, *scr):
    c = lax.axis_index("c")
    s = lax.axis_index("s")
    _seg_round(h_lo, h_hi, src2d, dst2d, zrows, sum_out, *scr, c, s)


_segsum2 = pl.kernel(
    _segsum2_body,
    out_type=(jax.ShapeDtypeStruct((NC * N_NODE, HALF), jnp.float32),
              jax.ShapeDtypeStruct((NC * N_NODE, HALF), jnp.float32)),
    mesh=plsc.VectorSubcoreMesh(**_sc_mesh),
    scratch_types=_seg_scratch(),
    compiler_params=_sc_params,
)

_segsum1 = pl.kernel(
    _segsum1_body,
    out_type=jax.ShapeDtypeStruct((NC * N_NODE, HALF), jnp.float32),
    mesh=plsc.VectorSubcoreMesh(**_sc_mesh),
    scratch_types=_seg_scratch(),
    compiler_params=_sc_params,
)


def _cnt_body(dstr2d, dstv2d, zcnt, cntr_out, cntv_out, cnt_v, idx_v):
    c = lax.axis_index("c")
    s = lax.axis_index("s")
    w = s * NC + c
    ones16 = jnp.full((16,), 1.0, jnp.float32)

    def _hist(dst2d, out):
        pltpu.sync_copy(zcnt, cnt_v)

        def body(i, carry):
            pltpu.sync_copy(dst2d.at[pl.ds(w * CR + i * CKB, CKB)], idx_v)
            for k in range(CKB):
                for t in range(CIB // 16):
                    idx16 = idx_v[k, pl.ds(t * 16, 16)]
                    plsc.addupdate_scatter(cnt_v, [idx16], ones16)
            return carry
        lax.fori_loop(0, CR // CKB, body, 0)

        @pl.when(w < CTAIL)
        def _tail():
            pltpu.sync_copy(dst2d.at[pl.ds(NW * CR + w, 1)],
                            idx_v.at[pl.ds(0, 1)])
            for t in range(CIB // 16):
                idx16 = idx_v[0, pl.ds(t * 16, 16)]
                plsc.addupdate_scatter(cnt_v, [idx16], ones16)

        pltpu.sync_copy(cnt_v, out.at[w])

    _hist(dstr2d, cntr_out)
    _hist(dstv2d, cntv_out)


_cnt_kernel = pl.kernel(
    _cnt_body,
    out_type=(jax.ShapeDtypeStruct((NW, N_NODE), jnp.float32),
              jax.ShapeDtypeStruct((NW, N_NODE), jnp.float32)),
    mesh=plsc.VectorSubcoreMesh(**_sc_mesh),
    scratch_types=[
        pltpu.VMEM((N_NODE,), jnp.float32),
        pltpu.VMEM((CKB, CIB), jnp.int32),
    ],
    compiler_params=_sc_params,
)


_RB = 2000


def _proj_relu(x, W, b):
    n, d = x.shape
    h = W.shape[0]

    def body(x_ref, w_ref, b_ref, o_ref):
        y = jnp.dot(x_ref[...], w_ref[...].T,
                    preferred_element_type=jnp.float32) + b_ref[...]
        o_ref[...] = jnp.maximum(y, 0.0)

    return pl.pallas_call(
        body,
        grid=(n // _RB,),
        in_specs=[
            pl.BlockSpec((_RB, d), lambda i: (i, 0)),
            pl.BlockSpec((h, d), lambda i: (0, 0)),
            pl.BlockSpec((1, h), lambda i: (0, 0)),
        ],
        out_specs=pl.BlockSpec((_RB, h), lambda i: (i, 0)),
        out_shape=jax.ShapeDtypeStruct((n, h), jnp.float32),
    )(x, W, b.reshape(1, h))


def _combine(sums, cntT, xdst, Wl, bl, Wr, relu, Wout=None, bout=None):
    n = xdst.shape[0]
    nb = n // _RB
    out_h = 2 if Wout is not None else H

    def body(slo_ref, shi_ref, cp_ref, xd_ref, wl_ref, bl_ref, wr_ref,
             *rest):
        if Wout is not None:
            wo_ref, bo_ref, o_ref = rest
        else:
            (o_ref,) = rest
        cnt = jnp.sum(cp_ref[...], axis=1)
        inv = 1.0 / jnp.maximum(cnt, 1.0)
        sm = jnp.concatenate([slo_ref[...], shi_ref[...]], axis=1)
        sm = sm * inv[:, None]
        y = (jnp.dot(sm, wl_ref[...].T, preferred_element_type=jnp.float32)
             + bl_ref[...]
             + jnp.dot(xd_ref[...], wr_ref[...].T,
                       preferred_element_type=jnp.float32))
        if relu:
            y = jnp.maximum(y, 0.0)
        if Wout is not None:
            y = jnp.dot(y, wo_ref[...].T,
                        preferred_element_type=jnp.float32) + bo_ref[...]
        o_ref[...] = y

    in_specs = [
        pl.BlockSpec((_RB, HALF), lambda i: (i, 0)),
        pl.BlockSpec((_RB, HALF), lambda i: (i + nb, 0)),
        pl.BlockSpec((_RB, NW), lambda i: (i, 0)),
        pl.BlockSpec((_RB, H), lambda i: (i, 0)),
        pl.BlockSpec((H, H), lambda i: (0, 0)),
        pl.BlockSpec((1, H), lambda i: (0, 0)),
        pl.BlockSpec((H, H), lambda i: (0, 0)),
    ]
    args = [sums, sums, cntT, xdst, Wl, bl.reshape(1, H), Wr]
    if Wout is not None:
        in_specs += [
            pl.BlockSpec((2, H), lambda i: (0, 0)),
            pl.BlockSpec((1, 2), lambda i: (0, 0)),
        ]
        args += [Wout, bout.reshape(1, 2)]

    return pl.pallas_call(
        body,
        grid=(nb,),
        in_specs=in_specs,
        out_specs=pl.BlockSpec((_RB, out_h), lambda i: (i, 0)),
        out_shape=jax.ShapeDtypeStruct((n, out_h), jnp.float32),
    )(*args)


def kernel(x_user, x_article, edge_index_reads, edge_index_rev, W_in_user,
           b_in_user, W_in_article, b_in_article, Wl1_reads, bl1_reads,
           Wr1_reads, Wl1_rev, bl1_rev, Wr1_rev, Wl2_reads, bl2_reads,
           Wr2_reads, Wl2_rev, bl2_rev, Wr2_rev, W_out, b_out):
    srcr2d = edge_index_reads[0].reshape(EROWS, IB)
    dstr2d = edge_index_reads[1].reshape(EROWS, IB)
    srcv2d = edge_index_rev[0].reshape(EROWS, IB)
    dstv2d = edge_index_rev[1].reshape(EROWS, IB)
    dstr128 = edge_index_reads[1].reshape(CROWS_T, CIB)
    dstv128 = edge_index_rev[1].reshape(CROWS_T, CIB)

    zcnt = jnp.zeros((N_NODE,), jnp.float32)
    zrows = jnp.zeros((FCH, HALF), jnp.float32)
    cntr, cntv = _cnt_kernel(dstr128, dstv128, zcnt)
    cntrT, cntvT = cntr.T, cntv.T

    h_u = _proj_relu(x_user, W_in_user, b_in_user)
    h_a = _proj_relu(x_article, W_in_article, b_in_article)

    sum1a, sum1u = _segsum2(h_u[:, :HALF], h_u[:, HALF:],
                            h_a[:, :HALF], h_a[:, HALF:],
                            srcr2d, dstr2d, srcv2d, dstv2d, zrows)
    h1_a = _combine(sum1a, cntrT, h_a, Wl1_reads, bl1_reads, Wr1_reads, True)
    h1_u = _combine(sum1u, cntvT, h_u, Wl1_rev, bl1_rev, Wr1_rev, True)

    sum2a = _segsum1(h1_u[:, :HALF], h1_u[:, HALF:], srcr2d, dstr2d, zrows)
    out = _combine(sum2a, cntrT, h1_a, Wl2_reads, bl2_reads, Wr2_reads,
                   False, Wout=W_out, bout=b_out)
    return out


# speedup vs baseline: 1.0340x; 1.0340x over previous
import jax
import jax.numpy as jnp
from jax import lax
from jax.experimental import pallas as pl
from jax.experimental.pallas import tpu as pltpu
from jax.experimental.pallas import tpu_sc as plsc

N_NODE = 50000
E = 800000
D_IN = 128
H = 64
HALF = H // 2
NC = 2
NS = 16
NW = NC * NS
IB = 100
KB = 2
EROWS = E // IB
RPS = EROWS // NS
NPAIR = RPS // (2 * KB)
ROWS_PT = N_NODE // NS
FCH = 100

CR = EROWS // NW
CKB = 5

_sc_params = pltpu.CompilerParams(use_tc_tiling_on_sc=False,
                                  needs_layout_passes=False)
_sc_mesh = dict(core_axis_name="c", subcore_axis_name="s")


def _seg_round(h3, src2d, dst2d, zrows, out_hbm, acc,
               srcA, dstA, srcB, dstB, rowsA, rowsB, gsA, gsB, c, s):
    pltpu.sync_copy(zrows, rowsA.at[0])
    for j in range(ROWS_PT // FCH):
        pltpu.sync_copy(rowsA.at[0],
                        acc.at[pl.ds(s * ROWS_PT + j * FCH, FCH)])
    pltpu.sync_copy(rowsA.at[0, pl.ds(0, 25)],
                    acc.at[pl.ds(s * ROWS_PT + 3100, 25)])
    plsc.subcore_barrier()

    def _accum(h_half):
        base = s * RPS

        def load_fire(row0, src_v, dst_v, rows_v, sem):
            pltpu.sync_copy(src2d.at[pl.ds(row0, KB)], src_v)
            pltpu.sync_copy(dst2d.at[pl.ds(row0, KB)], dst_v)
            for k in range(KB):
                pltpu.async_copy(h_half.at[src_v.at[k]], rows_v.at[k], sem)

        def wait_scatter(src_v, dst_v, rows_v, sem):
            for k in range(KB):
                pltpu.make_async_copy(h_half.at[src_v.at[k]],
                                      rows_v.at[k], sem).wait()
            for k in range(KB):
                pltpu.sync_copy(rows_v.at[k], acc.at[dst_v.at[k]], add=True)

        load_fire(base, srcA, dstA, rowsA, gsA)

        def step(j, carry):
            row0 = base + 2 * KB * j
            load_fire(row0 + KB, srcB, dstB, rowsB, gsB)
            wait_scatter(srcA, dstA, rowsA, gsA)

            @pl.when(j < NPAIR - 1)
            def _():
                load_fire(row0 + 2 * KB, srcA, dstA, rowsA, gsA)

            wait_scatter(srcB, dstB, rowsB, gsB)
            return carry
        lax.fori_loop(0, NPAIR, step, 0)

    @pl.when(c == 0)
    def _lo():
        _accum(h3.at[0])

    @pl.when(c == 1)
    def _hi():
        _accum(h3.at[1])

    plsc.subcore_barrier()

    for j in range(ROWS_PT // FCH):
        r0 = s * ROWS_PT + j * FCH
        pltpu.sync_copy(acc.at[pl.ds(r0, FCH)], rowsA.at[0])
        pltpu.sync_copy(rowsA.at[0],
                        out_hbm.at[pl.ds(r0, FCH), pl.ds(c * HALF, HALF)])
    r0 = s * ROWS_PT + 3100
    pltpu.sync_copy(acc.at[pl.ds(r0, 25)], rowsA.at[0, pl.ds(0, 25)])
    pltpu.sync_copy(rowsA.at[0, pl.ds(0, 25)],
                    out_hbm.at[pl.ds(r0, 25), pl.ds(c * HALF, HALF)])


def _seg_scratch():
    return [
        pltpu.VMEM_SHARED((N_NODE, HALF), jnp.float32),
        pltpu.VMEM((KB, IB), jnp.int32),
        pltpu.VMEM((KB, IB), jnp.int32),
        pltpu.VMEM((KB, IB), jnp.int32),
        pltpu.VMEM((KB, IB), jnp.int32),
        pltpu.VMEM((KB, IB, HALF), jnp.float32),
        pltpu.VMEM((KB, IB, HALF), jnp.float32),
        pltpu.SemaphoreType.DMA,
        pltpu.SemaphoreType.DMA,
    ]


def _segsum2_body(hu3, ha3, srcr, dstr, srcv, dstv, zrows,
                  sum_r, sum_v, *scr):
    c = lax.axis_index("c")
    s = lax.axis_index("s")
    _seg_round(hu3, srcr, dstr, zrows, sum_r, *scr, c, s)
    _seg_round(ha3, srcv, dstv, zrows, sum_v, *scr, c, s)


def _segsum1_body(h3, src2d, dst2d, zrows, sum_out, *scr):
    c = lax.axis_index("c")
    s = lax.axis_index("s")
    _seg_round(h3, src2d, dst2d, zrows, sum_out, *scr, c, s)


_segsum2 = pl.kernel(
    _segsum2_body,
    out_type=(jax.ShapeDtypeStruct((N_NODE, H), jnp.float32),
              jax.ShapeDtypeStruct((N_NODE, H), jnp.float32)),
    mesh=plsc.VectorSubcoreMesh(**_sc_mesh),
    scratch_types=_seg_scratch(),
    compiler_params=_sc_params,
)

_segsum1 = pl.kernel(
    _segsum1_body,
    out_type=jax.ShapeDtypeStruct((N_NODE, H), jnp.float32),
    mesh=plsc.VectorSubcoreMesh(**_sc_mesh),
    scratch_types=_seg_scratch(),
    compiler_params=_sc_params,
)


def _cnt_body(dstr2d, dstv2d, zcnt, cntr_out, cntv_out, cnt_v, idx_v):
    c = lax.axis_index("c")
    s = lax.axis_index("s")
    w = s * NC + c
    ones16 = jnp.full((16,), 1.0, jnp.float32)
    tailmask = lax.iota(jnp.int32, 16) >= (16 - (IB - (IB // 16) * 16))

    def _hist(dst2d, out):
        pltpu.sync_copy(zcnt, cnt_v)

        def body(i, carry):
            pltpu.sync_copy(dst2d.at[pl.ds(w * CR + i * CKB, CKB)], idx_v)
            for k in range(CKB):
                for t in range(IB // 16):
                    idx16 = idx_v[k, pl.ds(t * 16, 16)]
                    plsc.addupdate_scatter(cnt_v, [idx16], ones16)
                idxt = idx_v[k, pl.ds(IB - 16, 16)]
                plsc.addupdate_scatter(cnt_v, [idxt], ones16, mask=tailmask)
            return carry
        lax.fori_loop(0, CR // CKB, body, 0)

        pltpu.sync_copy(cnt_v, out.at[w])

    _hist(dstr2d, cntr_out)
    _hist(dstv2d, cntv_out)


_cnt_kernel = pl.kernel(
    _cnt_body,
    out_type=(jax.ShapeDtypeStruct((NW, N_NODE), jnp.float32),
              jax.ShapeDtypeStruct((NW, N_NODE), jnp.float32)),
    mesh=plsc.VectorSubcoreMesh(**_sc_mesh),
    scratch_types=[
        pltpu.VMEM((N_NODE,), jnp.float32),
        pltpu.VMEM((CKB, IB), jnp.int32),
    ],
    compiler_params=_sc_params,
)


_RB = 2000


def _proj_relu(x, W, b):
    n, d = x.shape

    def body(x_ref, w_ref, b_ref, o_ref):
        y = jnp.dot(x_ref[...], w_ref[...].T,
                    preferred_element_type=jnp.float32) + b_ref[0]
        o_ref[...] = jnp.maximum(y, 0.0)[None]

    return pl.pallas_call(
        body,
        grid=(n // _RB, 2),
        in_specs=[
            pl.BlockSpec((_RB, d), lambda i, j: (i, 0)),
            pl.BlockSpec((HALF, d), lambda i, j: (j, 0)),
            pl.BlockSpec((1, 1, HALF), lambda i, j: (j, 0, 0)),
        ],
        out_specs=pl.BlockSpec((1, _RB, HALF), lambda i, j: (j, i, 0)),
        out_shape=jax.ShapeDtypeStruct((2, n, HALF), jnp.float32),
    )(x, W, b.reshape(2, 1, HALF))


def _combine(sums, cntT, xd3, Wl, bl, Wr, relu, out3d, Wout=None, bout=None):
    n = xd3.shape[1]
    nb = n // _RB

    def body(s_ref, cp_ref, xlo_ref, xhi_ref, wl_ref, bl_ref, wr_ref,
             *rest):
        if Wout is not None:
            wo_ref, bo_ref, o_ref = rest
        else:
            (o_ref,) = rest
        cnt = jnp.sum(cp_ref[...], axis=1)
        inv = 1.0 / jnp.maximum(cnt, 1.0)
        sm = s_ref[...] * inv[:, None]
        xd = jnp.concatenate([xlo_ref[0], xhi_ref[0]], axis=1)
        blv = bl_ref[0] if out3d else bl_ref[...]
        y = (jnp.dot(sm, wl_ref[...].T, preferred_element_type=jnp.float32)
             + blv
             + jnp.dot(xd, wr_ref[...].T,
                       preferred_element_type=jnp.float32))
        if relu:
            y = jnp.maximum(y, 0.0)
        if Wout is not None:
            y = jnp.dot(y, wo_ref[...].T,
                        preferred_element_type=jnp.float32) + bo_ref[...]
        o_ref[...] = y[None] if out3d else y

    grid = (nb, 2) if out3d else (nb,)
    if out3d:
        ix = lambda i, j: (i, 0)
        wl_spec = pl.BlockSpec((HALF, H), lambda i, j: (j, 0))
        bl_spec = pl.BlockSpec((1, 1, HALF), lambda i, j: (j, 0, 0))
        wr_spec = pl.BlockSpec((HALF, H), lambda i, j: (j, 0))
        out_spec = pl.BlockSpec((1, _RB, HALF), lambda i, j: (j, i, 0))
        out_sh = (2, n, HALF)
    else:
        ix = lambda i: (i, 0)
        wl_spec = pl.BlockSpec((H, H), lambda i: (0, 0))
        bl_spec = pl.BlockSpec((1, H), lambda i: (0, 0))
        wr_spec = pl.BlockSpec((H, H), lambda i: (0, 0))
        out_h = 2 if Wout is not None else H
        out_spec = pl.BlockSpec((_RB, out_h), lambda i: (i, 0))
        out_sh = (n, out_h)

    xlo_spec = pl.BlockSpec((1, _RB, HALF),
                            (lambda i, j: (0, i, 0)) if out3d
                            else (lambda i: (0, i, 0)))
    xhi_spec = pl.BlockSpec((1, _RB, HALF),
                            (lambda i, j: (1, i, 0)) if out3d
                            else (lambda i: (1, i, 0)))
    in_specs = [
        pl.BlockSpec((_RB, H), ix),
        pl.BlockSpec((_RB, NW), ix),
        xlo_spec, xhi_spec, wl_spec, bl_spec, wr_spec,
    ]
    args = [sums, cntT, xd3, xd3, Wl,
            bl.reshape(2, 1, HALF) if out3d else bl.reshape(1, H), Wr]
    if Wout is not None:
        in_specs += [
            pl.BlockSpec((2, H), lambda i: (0, 0)),
            pl.BlockSpec((1, 2), lambda i: (0, 0)),
        ]
        args += [Wout, bout.reshape(1, 2)]

    return pl.pallas_call(
        body,
        grid=grid,
        in_specs=in_specs,
        out_specs=out_spec,
        out_shape=jax.ShapeDtypeStruct(out_sh, jnp.float32),
    )(*args)


def kernel(x_user, x_article, edge_index_reads, edge_index_rev, W_in_user,
           b_in_user, W_in_article, b_in_article, Wl1_reads, bl1_reads,
           Wr1_reads, Wl1_rev, bl1_rev, Wr1_rev, Wl2_reads, bl2_reads,
           Wr2_reads, Wl2_rev, bl2_rev, Wr2_rev, W_out, b_out):
    srcr2d = edge_index_reads[0].reshape(EROWS, IB)
    dstr2d = edge_index_reads[1].reshape(EROWS, IB)
    srcv2d = edge_index_rev[0].reshape(EROWS, IB)
    dstv2d = edge_index_rev[1].reshape(EROWS, IB)

    zcnt = jnp.zeros((N_NODE,), jnp.float32)
    zrows = jnp.zeros((FCH, HALF), jnp.float32)
    cntr, cntv = _cnt_kernel(dstr2d, dstv2d, zcnt)
    cntrT, cntvT = cntr.T, cntv.T

    hu3 = _proj_relu(x_user, W_in_user, b_in_user)
    ha3 = _proj_relu(x_article, W_in_article, b_in_article)

    sum1a, sum1u = _segsum2(hu3, ha3, srcr2d, dstr2d, srcv2d, dstv2d, zrows)
    h1a3 = _combine(sum1a, cntrT, ha3, Wl1_reads, bl1_reads, Wr1_reads,
                    True, True)
    h1u3 = _combine(sum1u, cntvT, hu3, Wl1_rev, bl1_rev, Wr1_rev,
                    True, True)

    sum2a = _segsum1(h1u3, srcr2d, dstr2d, zrows)
    out = _combine(sum2a, cntrT, h1a3, Wl2_reads, bl2_reads,
                   Wr2_reads, False, False, Wout=W_out, bout=b_out)
    return out
